# baseline (speedup 1.0000x reference)
import functools

import jax
import jax.numpy as jnp
from jax import lax
from jax.experimental import pallas as pl
from jax.experimental.pallas import tpu as pltpu
from jax.experimental.pallas import tpu_sc as plsc

EPS = 1e-5

NC = 2
NS = 16
CH = 128
CHA = 64
IB = 4
MB = 400


def _elu(v):
    return jnp.where(v > 0, v, jnp.exp(jnp.minimum(v, 0.0)) - 1.0)


def _mm_body(has_skip, RH, x_ref, w_ref, tbl_ref, root_ref, *rest):
    acc = jnp.dot(x_ref[...], w_ref[...], preferred_element_type=jnp.float32)
    tbl_ref[0] = acc[:, :RH]
    tbl_ref[1] = acc[:, RH:2 * RH]
    root_ref[...] = acc[:, 2 * RH:2 * RH + 256]
    if has_skip:
        rest[0][...] = acc[:, 2 * RH + 256:2 * RH + 512]


def _mm_call(xb, wcat, N, R, H, has_skip):
    RH = R * H
    KW = wcat.shape[1]
    grid = N // MB
    outs = [
        jax.ShapeDtypeStruct((NC, N, RH), jnp.float32),
        jax.ShapeDtypeStruct((N, 256), jnp.float32),
    ]
    out_specs = [
        pl.BlockSpec((NC, MB, RH), lambda i: (0, i, 0)),
        pl.BlockSpec((MB, 256), lambda i: (i, 0)),
    ]
    if has_skip:
        outs.append(jax.ShapeDtypeStruct((N, 256), jnp.float32))
        out_specs.append(pl.BlockSpec((MB, 256), lambda i: (i, 0)))
    return pl.pallas_call(
        functools.partial(_mm_body, has_skip, RH),
        grid=(grid,),
        in_specs=[
            pl.BlockSpec((MB, xb.shape[1]), lambda i: (i, 0)),
            pl.BlockSpec((xb.shape[1], KW), lambda i: (0, 0)),
        ],
        out_specs=out_specs,
        out_shape=outs,
    )(xb, wcat)


def _make_sc_agg(N, R, H, ACC, ETP):
    NCH = ETP // CHA
    NBLK = NCH // IB
    rows_per = ACC // NS
    mesh = plsc.VectorSubcoreMesh(core_axis_name="c", subcore_axis_name="s")

    def body(table, idx_hbm, zacc, out, acc_sh, ibuf, b0, b1, b2, b3,
             si0, si1, sg0, sg1, sg2, sg3):
        bufs = [b0, b1, b2, b3]
        sgs = [sg0, sg1, sg2, sg3]
        sis = [si0, si1]
        cid = lax.axis_index("c")
        sid = lax.axis_index("s")
        r0 = sid * rows_per
        pltpu.sync_copy(zacc.at[pl.ds(r0, rows_per)],
                        acc_sh.at[pl.ds(r0, rows_per)])
        plsc.subcore_barrier()

        tidx = idx_hbm.at[sid]

        pltpu.sync_copy(tidx.at[pl.ds(0, IB)], ibuf.at[0])
        pltpu.async_copy(tidx.at[pl.ds(IB, IB)], ibuf.at[1], sis[1])
        for b in range(IB):
            pltpu.async_copy(table.at[cid].at[ibuf.at[0].at[b].at[0]],
                             bufs[b], sgs[b])

        def outer(kk, carry):
            for s in range(2):
                k = 2 * kk + s
                pltpu.make_async_copy(tidx.at[pl.ds(0, IB)],
                                      ibuf.at[1 - s], sis[1 - s]).wait()
                for b in range(IB):
                    pltpu.make_async_copy(table.at[cid].at[pl.ds(0, CHA)],
                                          bufs[b], sgs[b]).wait()
                    pltpu.sync_copy(bufs[b],
                                    acc_sh.at[ibuf.at[s].at[b].at[1]],
                                    add=True)
                    pltpu.async_copy(
                        table.at[cid].at[ibuf.at[1 - s].at[b].at[0]],
                        bufs[b], sgs[b])
                pltpu.async_copy(tidx.at[pl.ds((k + 2) * IB, IB)],
                                 ibuf.at[s], sis[s])
            return carry

        lax.fori_loop(0, NBLK // 2, outer, 0)
        for b in range(IB):
            pltpu.make_async_copy(table.at[cid].at[pl.ds(0, CHA)],
                                  bufs[b], sgs[b]).wait()
        pltpu.make_async_copy(tidx.at[pl.ds(0, IB)], ibuf.at[1],
                              sis[1]).wait()
        plsc.subcore_barrier()
        pltpu.sync_copy(acc_sh.at[pl.ds(r0, rows_per)],
                        out.at[cid].at[pl.ds(r0, rows_per)])

    scratch = [
        pltpu.VMEM_SHARED((ACC, H), jnp.float32),
        pltpu.VMEM((2, IB, 2, CHA), jnp.int32),
        pltpu.VMEM((CHA, H), jnp.float32),
        pltpu.VMEM((CHA, H), jnp.float32),
        pltpu.VMEM((CHA, H), jnp.float32),
        pltpu.VMEM((CHA, H), jnp.float32),
        pltpu.SemaphoreType.DMA,
        pltpu.SemaphoreType.DMA,
        pltpu.SemaphoreType.DMA,
        pltpu.SemaphoreType.DMA,
        pltpu.SemaphoreType.DMA,
        pltpu.SemaphoreType.DMA,
    ]
    return pl.kernel(body, out_type=jax.ShapeDtypeStruct((NC, ACC, H),
                                                         jnp.float32),
                     mesh=mesh, scratch_types=scratch)


def _make_sc_cnt(ACC, ETC):
    NCHC = ETC // CH
    rows_per = ACC // NS
    mesh = plsc.VectorSubcoreMesh(core_axis_name="c", subcore_axis_name="s")

    def body(didx, zcnt, ones, cnt_out, cnt_sh, ibuf, ones_v):
        cid = lax.axis_index("c")
        sid = lax.axis_index("s")
        wid = cid * NS + sid
        r0 = sid * rows_per
        pltpu.sync_copy(zcnt.at[pl.ds(r0, rows_per)],
                        cnt_sh.at[pl.ds(r0, rows_per)])
        pltpu.sync_copy(ones, ones_v)
        plsc.subcore_barrier()

        def step(t, carry):
            pltpu.sync_copy(didx.at[wid].at[pl.ds(2 * t, 2)], ibuf)
            pltpu.sync_copy(ones_v, cnt_sh.at[ibuf.at[0].at[0]], add=True)
            pltpu.sync_copy(ones_v, cnt_sh.at[ibuf.at[1].at[0]], add=True)
            return carry

        lax.fori_loop(0, NCHC // 2, step, 0)
        plsc.subcore_barrier()
        pltpu.sync_copy(cnt_sh.at[pl.ds(r0, rows_per)],
                        cnt_out.at[cid].at[pl.ds(r0, rows_per)])

    scratch = [
        pltpu.VMEM_SHARED((ACC, CH), jnp.float32),
        pltpu.VMEM((2, 1, CH), jnp.int32),
        pltpu.VMEM((CH, CH), jnp.float32),
    ]
    return pl.kernel(body, out_type=jax.ShapeDtypeStruct((NC, ACC, CH),
                                                         jnp.float32),
                     mesh=mesh, scratch_types=scratch)


def _stage_a_body(N, a0_ref, a1_ref, cnt_ref, root_ref, b_ref,
                  hpre_ref, stats_ref):
    i = pl.program_id(0)
    cnt = jnp.maximum(jnp.sum(cnt_ref[...], axis=(0, 2)) * (1.0 / 128.0), 1.0)
    h = (jnp.concatenate([a0_ref[0], a1_ref[0]], axis=1) / cnt[:, None]
         + root_ref[...] + b_ref[...])
    hpre_ref[...] = h
    s = jnp.concatenate([jnp.sum(h, axis=0, keepdims=True),
                         jnp.sum(h * h, axis=0, keepdims=True)], axis=0)

    @pl.when(i == 0)
    def _():
        stats_ref[...] = s

    @pl.when(i > 0)
    def _():
        stats_ref[...] += s


def _stage_b_body(N, hpre_ref, stats_ref, g_ref, be_ref, out_ref):
    s = stats_ref[...]
    mean = s[0:1] * (1.0 / N)
    var = s[1:2] * (1.0 / N) - mean * mean
    inv = lax.rsqrt(var + EPS) * g_ref[...]
    y = (hpre_ref[...] - mean) * inv + be_ref[...]
    out_ref[...] = _elu(y).astype(jnp.bfloat16)


def _stage_c_body(a0_ref, a1_ref, cnt_ref, root_ref, b_ref, skip_ref,
                  sb_ref, out_ref):
    cnt = jnp.maximum(jnp.sum(cnt_ref[...], axis=(0, 2)) * (1.0 / 128.0), 1.0)
    h = (jnp.concatenate([a0_ref[0], a1_ref[0]], axis=1) / cnt[:, None]
         + root_ref[...] + b_ref[...])
    h = _elu(h)
    h = h + skip_ref[...] + sb_ref[...]
    out_ref[...] = _elu(h)


def _half_spec(c):
    return pl.BlockSpec((1, MB, 128), lambda i, c=c: (c, i, 0))


def kernel(x, edge_index, edge_types, w0, root0, b0, w1, root1, b1,
           gamma0, beta0, skip_w, skip_b):
    N, F = x.shape
    R = w0.shape[0]
    E = edge_index.shape[1]
    H = F // 2
    RH = R * H

    ETP = -(-E // NS // (2 * IB * CHA)) * (2 * IB * CHA)
    EP = NS * ETP
    ACC = -(-(N + 1) // (NS * 8)) * (NS * 8)
    NCH = ETP // CHA

    src = edge_index[0].astype(jnp.int32)
    dst = edge_index[1].astype(jnp.int32)
    ety = edge_types.astype(jnp.int32)
    pad = EP - E
    src_p = jnp.concatenate([src, jnp.zeros((pad,), jnp.int32)])
    ety_p = jnp.concatenate([ety, jnp.zeros((pad,), jnp.int32)])
    dst_p = jnp.concatenate([dst, jnp.full((pad,), N, jnp.int32)])
    gidx = (src_p * R + ety_p).reshape(NS, NCH, CHA)
    dst2 = dst_p.reshape(NS, NCH, CHA)
    idx_all = jnp.stack([gidx, dst2], axis=2)
    idx_pad = jnp.stack(
        [jnp.zeros((NS, 2 * IB, CHA), jnp.int32),
         jnp.full((NS, 2 * IB, CHA), N, jnp.int32)], axis=2)
    idx_all = jnp.concatenate([idx_all, idx_pad], axis=1)
    ETC = EP // (NC * NS)
    didx = dst_p.reshape(NC * NS, ETC // CH, 1, CH)
    zacc = jnp.zeros((ACC, H), jnp.float32)
    zcnt = jnp.zeros((ACC, CH), jnp.float32)
    ones = jnp.ones((CH, CH), jnp.float32)

    def wcat_of(w, extra):
        h0 = w[:, :, :H].transpose(1, 0, 2).reshape(F, RH)
        h1 = w[:, :, H:].transpose(1, 0, 2).reshape(F, RH)
        return jnp.concatenate([h0, h1] + extra, axis=1).astype(jnp.bfloat16)

    wcat0 = wcat_of(w0, [root0, skip_w])
    wcat1 = wcat_of(w1, [root1])
    xb = x.astype(jnp.bfloat16)
    b0r = b0.reshape(1, 256)
    b1r = b1.reshape(1, 256)
    g0r = gamma0.reshape(1, 256)
    be0r = beta0.reshape(1, 256)
    sbr = skip_b.reshape(1, 256)

    sc_agg = _make_sc_agg(N, R, H, ACC, ETP)
    sc_cnt = _make_sc_cnt(ACC, ETC)

    cnt_p = sc_cnt(didx, zcnt, ones)
    tbl0, xroot0, xskip = _mm_call(xb, wcat0, N, R, H, True)
    agg0 = sc_agg(tbl0.reshape(NC, N * R, H), idx_all, zacc)

    grid = N // MB
    hpre, stats = pl.pallas_call(
        functools.partial(_stage_a_body, N),
        grid=(grid,),
        in_specs=[
            _half_spec(0),
            _half_spec(1),
            pl.BlockSpec((NC, MB, 128), lambda i: (0, i, 0)),
            pl.BlockSpec((MB, 256), lambda i: (i, 0)),
            pl.BlockSpec((1, 256), lambda i: (0, 0)),
        ],
        out_specs=[
            pl.BlockSpec((MB, 256), lambda i: (i, 0)),
            pl.BlockSpec((2, 256), lambda i: (0, 0)),
        ],
        out_shape=[
            jax.ShapeDtypeStruct((N, 256), jnp.float32),
            jax.ShapeDtypeStruct((2, 256), jnp.float32),
        ],
    )(agg0, agg0, cnt_p, xroot0, b0r)

    h0b = pl.pallas_call(
        functools.partial(_stage_b_body, N),
        grid=(grid,),
        in_specs=[
            pl.BlockSpec((MB, 256), lambda i: (i, 0)),
            pl.BlockSpec((2, 256), lambda i: (0, 0)),
            pl.BlockSpec((1, 256), lambda i: (0, 0)),
            pl.BlockSpec((1, 256), lambda i: (0, 0)),
        ],
        out_specs=pl.BlockSpec((MB, 256), lambda i: (i, 0)),
        out_shape=jax.ShapeDtypeStruct((N, 256), jnp.bfloat16),
    )(hpre, stats, g0r, be0r)

    tbl1, hroot1 = _mm_call(h0b, wcat1, N, R, H, False)
    agg1 = sc_agg(tbl1.reshape(NC, N * R, H), idx_all, zacc)

    out = pl.pallas_call(
        _stage_c_body,
        grid=(grid,),
        in_specs=[
            _half_spec(0),
            _half_spec(1),
            pl.BlockSpec((NC, MB, 128), lambda i: (0, i, 0)),
            pl.BlockSpec((MB, 256), lambda i: (i, 0)),
            pl.BlockSpec((1, 256), lambda i: (0, 0)),
            pl.BlockSpec((MB, 256), lambda i: (i, 0)),
            pl.BlockSpec((1, 256), lambda i: (0, 0)),
        ],
        out_specs=pl.BlockSpec((MB, 256), lambda i: (i, 0)),
        out_shape=jax.ShapeDtypeStruct((N, 256), jnp.float32),
    )(agg1, agg1, cnt_p, hroot1, b1r, xskip, sbr)
    return out


# speedup vs baseline: 1.0106x; 1.0106x over previous
import functools

import jax
import jax.numpy as jnp
from jax import lax
from jax.experimental import pallas as pl
from jax.experimental.pallas import tpu as pltpu
from jax.experimental.pallas import tpu_sc as plsc

EPS = 1e-5

NC = 2
NS = 16
CH = 128
IB = 4
MB = 400


def _elu(v):
    return jnp.where(v > 0, v, jnp.exp(jnp.minimum(v, 0.0)) - 1.0)


def _mm_body(has_skip, RH, x_ref, w_ref, tbl_ref, root_ref, *rest):
    acc = jnp.dot(x_ref[...], w_ref[...], preferred_element_type=jnp.float32)
    tbl_ref[0] = acc[:, :RH]
    tbl_ref[1] = acc[:, RH:2 * RH]
    root_ref[...] = acc[:, 2 * RH:2 * RH + 256]
    if has_skip:
        rest[0][...] = acc[:, 2 * RH + 256:2 * RH + 512]


def _mm_call(xb, wcat, N, R, has_skip):
    RH = R * 128
    KW = wcat.shape[1]
    grid = N // MB
    outs = [
        jax.ShapeDtypeStruct((NC, N, RH), jnp.float32),
        jax.ShapeDtypeStruct((N, 256), jnp.float32),
    ]
    out_specs = [
        pl.BlockSpec((NC, MB, RH), lambda i: (0, i, 0)),
        pl.BlockSpec((MB, 256), lambda i: (i, 0)),
    ]
    if has_skip:
        outs.append(jax.ShapeDtypeStruct((N, 256), jnp.float32))
        out_specs.append(pl.BlockSpec((MB, 256), lambda i: (i, 0)))
    return pl.pallas_call(
        functools.partial(_mm_body, has_skip, RH),
        grid=(grid,),
        in_specs=[
            pl.BlockSpec((MB, xb.shape[1]), lambda i: (i, 0)),
            pl.BlockSpec((xb.shape[1], KW), lambda i: (0, 0)),
        ],
        out_specs=out_specs,
        out_shape=outs,
    )(xb, wcat)


def _make_sc_agg(N, R, ACC, ETP):
    NCH = ETP // CH
    NBLK = NCH // IB
    rows_per = ACC // NS
    mesh = plsc.VectorSubcoreMesh(core_axis_name="c", subcore_axis_name="s")

    def body(table, idx_hbm, zacc, out, acc_sh, ibuf, g0, g1,
             si0, si1, sg0, sg1):
        gbufs = [g0, g1]
        sgs = [sg0, sg1]
        sis = [si0, si1]
        cid = lax.axis_index("c")
        sid = lax.axis_index("s")
        r0 = sid * rows_per
        pltpu.sync_copy(zacc.at[pl.ds(r0, rows_per)],
                        acc_sh.at[pl.ds(r0, rows_per)])
        plsc.subcore_barrier()

        tidx = idx_hbm.at[sid]
        tbl_c = table.at[cid]

        pltpu.sync_copy(tidx.at[pl.ds(0, IB)], ibuf.at[0])
        pltpu.async_copy(tidx.at[pl.ds(IB, IB)], ibuf.at[1], sis[1])
        for b in range(2):
            pltpu.async_copy(tbl_c.at[ibuf.at[0].at[b].at[0]],
                             gbufs[b], sgs[b])

        def outer(kk, carry):
            for s in range(2):
                k = 2 * kk + s
                pltpu.make_async_copy(tidx.at[pl.ds(0, IB)],
                                      ibuf.at[1 - s], sis[1 - s]).wait()
                for b in range(IB):
                    p = b % 2
                    pltpu.make_async_copy(tbl_c.at[pl.ds(0, CH)],
                                          gbufs[p], sgs[p]).wait()
                    pltpu.sync_copy(gbufs[p],
                                    acc_sh.at[ibuf.at[s].at[b].at[1]],
                                    add=True)
                    if b < 2:
                        nxt = ibuf.at[s].at[b + 2].at[0]
                    else:
                        nxt = ibuf.at[1 - s].at[b - 2].at[0]
                    pltpu.async_copy(tbl_c.at[nxt], gbufs[p], sgs[p])
                pltpu.async_copy(tidx.at[pl.ds((k + 2) * IB, IB)],
                                 ibuf.at[s], sis[s])
            return carry

        lax.fori_loop(0, NBLK // 2, outer, 0)
        for b in range(2):
            pltpu.make_async_copy(tbl_c.at[pl.ds(0, CH)],
                                  gbufs[b], sgs[b]).wait()
        pltpu.make_async_copy(tidx.at[pl.ds(0, IB)], ibuf.at[1],
                              sis[1]).wait()
        plsc.subcore_barrier()
        pltpu.sync_copy(acc_sh.at[pl.ds(r0, rows_per)],
                        out.at[cid].at[pl.ds(r0, rows_per)])

    scratch = [
        pltpu.VMEM_SHARED((ACC, 128), jnp.float32),
        pltpu.VMEM((2, IB, 2, CH), jnp.int32),
        pltpu.VMEM((CH, 128), jnp.float32),
        pltpu.VMEM((CH, 128), jnp.float32),
        pltpu.SemaphoreType.DMA,
        pltpu.SemaphoreType.DMA,
        pltpu.SemaphoreType.DMA,
        pltpu.SemaphoreType.DMA,
    ]
    return pl.kernel(body, out_type=jax.ShapeDtypeStruct((NC, ACC, 128),
                                                         jnp.float32),
                     mesh=mesh, scratch_types=scratch)


def _make_sc_cnt(ACC, ETC):
    NCHC = ETC // CH
    rows_per = ACC // NS
    mesh = plsc.VectorSubcoreMesh(core_axis_name="c", subcore_axis_name="s")

    def body(didx, zcnt, ones, cnt_out, cnt_sh, ibuf, ones_v):
        cid = lax.axis_index("c")
        sid = lax.axis_index("s")
        wid = cid * NS + sid
        r0 = sid * rows_per
        pltpu.sync_copy(zcnt.at[pl.ds(r0, rows_per)],
                        cnt_sh.at[pl.ds(r0, rows_per)])
        pltpu.sync_copy(ones, ones_v)
        plsc.subcore_barrier()

        def step(t, carry):
            pltpu.sync_copy(didx.at[wid].at[pl.ds(2 * t, 2)], ibuf)
            pltpu.sync_copy(ones_v, cnt_sh.at[ibuf.at[0].at[0]], add=True)
            pltpu.sync_copy(ones_v, cnt_sh.at[ibuf.at[1].at[0]], add=True)
            return carry

        lax.fori_loop(0, NCHC // 2, step, 0)
        plsc.subcore_barrier()
        pltpu.sync_copy(cnt_sh.at[pl.ds(r0, rows_per)],
                        cnt_out.at[cid].at[pl.ds(r0, rows_per)])

    scratch = [
        pltpu.VMEM_SHARED((ACC, CH), jnp.float32),
        pltpu.VMEM((2, 1, CH), jnp.int32),
        pltpu.VMEM((CH, CH), jnp.float32),
    ]
    return pl.kernel(body, out_type=jax.ShapeDtypeStruct((NC, ACC, CH),
                                                         jnp.float32),
                     mesh=mesh, scratch_types=scratch)


def _stage_a_body(N, a0_ref, a1_ref, cnt_ref, root_ref, b_ref,
                  hpre_ref, stats_ref):
    i = pl.program_id(0)
    cnt = jnp.maximum(jnp.sum(cnt_ref[...], axis=(0, 2)) * (1.0 / 128.0), 1.0)
    h = (jnp.concatenate([a0_ref[0], a1_ref[0]], axis=1) / cnt[:, None]
         + root_ref[...] + b_ref[...])
    hpre_ref[...] = h
    s = jnp.concatenate([jnp.sum(h, axis=0, keepdims=True),
                         jnp.sum(h * h, axis=0, keepdims=True)], axis=0)

    @pl.when(i == 0)
    def _():
        stats_ref[...] = s

    @pl.when(i > 0)
    def _():
        stats_ref[...] += s


def _stage_b_body(N, hpre_ref, stats_ref, g_ref, be_ref, out_ref):
    s = stats_ref[...]
    mean = s[0:1] * (1.0 / N)
    var = s[1:2] * (1.0 / N) - mean * mean
    inv = lax.rsqrt(var + EPS) * g_ref[...]
    y = (hpre_ref[...] - mean) * inv + be_ref[...]
    out_ref[...] = _elu(y).astype(jnp.bfloat16)


def _stage_c_body(a0_ref, a1_ref, cnt_ref, root_ref, b_ref, skip_ref,
                  sb_ref, out_ref):
    cnt = jnp.maximum(jnp.sum(cnt_ref[...], axis=(0, 2)) * (1.0 / 128.0), 1.0)
    h = (jnp.concatenate([a0_ref[0], a1_ref[0]], axis=1) / cnt[:, None]
         + root_ref[...] + b_ref[...])
    h = _elu(h)
    h = h + skip_ref[...] + sb_ref[...]
    out_ref[...] = _elu(h)


def _half_spec(c):
    return pl.BlockSpec((1, MB, 128), lambda i, c=c: (c, i, 0))


def kernel(x, edge_index, edge_types, w0, root0, b0, w1, root1, b1,
           gamma0, beta0, skip_w, skip_b):
    N, F = x.shape
    R = w0.shape[0]
    E = edge_index.shape[1]

    NW = NC * NS
    ETP = -(-E // NS // (2 * IB * CH)) * (2 * IB * CH)
    EP = NS * ETP
    ACC = -(-(N + 1) // (NS * 8)) * (NS * 8)
    NCH = ETP // CH

    src = edge_index[0].astype(jnp.int32)
    dst = edge_index[1].astype(jnp.int32)
    ety = edge_types.astype(jnp.int32)
    pad = EP - E
    src_p = jnp.concatenate([src, jnp.zeros((pad,), jnp.int32)])
    ety_p = jnp.concatenate([ety, jnp.zeros((pad,), jnp.int32)])
    dst_p = jnp.concatenate([dst, jnp.full((pad,), N, jnp.int32)])
    gidx = (src_p * R + ety_p).reshape(NS, NCH, CH)
    dst2 = dst_p.reshape(NS, NCH, CH)
    idx_all = jnp.stack([gidx, dst2], axis=2)
    idx_pad = jnp.stack(
        [jnp.zeros((NS, 2 * IB, CH), jnp.int32),
         jnp.full((NS, 2 * IB, CH), N, jnp.int32)], axis=2)
    idx_all = jnp.concatenate([idx_all, idx_pad], axis=1)
    ETC = EP // NW
    didx = dst_p.reshape(NW, ETC // CH, 1, CH)
    zacc = jnp.zeros((ACC, 128), jnp.float32)
    zcnt = jnp.zeros((ACC, CH), jnp.float32)
    ones = jnp.ones((CH, CH), jnp.float32)

    def wcat_of(w, extra):
        h0 = w[:, :, :128].transpose(1, 0, 2).reshape(F, R * 128)
        h1 = w[:, :, 128:].transpose(1, 0, 2).reshape(F, R * 128)
        return jnp.concatenate([h0, h1] + extra, axis=1).astype(jnp.bfloat16)

    wcat0 = wcat_of(w0, [root0, skip_w])
    wcat1 = wcat_of(w1, [root1])
    xb = x.astype(jnp.bfloat16)
    b0r = b0.reshape(1, 256)
    b1r = b1.reshape(1, 256)
    g0r = gamma0.reshape(1, 256)
    be0r = beta0.reshape(1, 256)
    sbr = skip_b.reshape(1, 256)

    sc_agg = _make_sc_agg(N, R, ACC, ETP)
    sc_cnt = _make_sc_cnt(ACC, ETC)

    cnt_p = sc_cnt(didx, zcnt, ones)
    tbl0, xroot0, xskip = _mm_call(xb, wcat0, N, R, True)
    agg0 = sc_agg(tbl0.reshape(NC, N * R, 128), idx_all, zacc)

    grid = N // MB
    hpre, stats = pl.pallas_call(
        functools.partial(_stage_a_body, N),
        grid=(grid,),
        in_specs=[
            _half_spec(0),
            _half_spec(1),
            pl.BlockSpec((NC, MB, 128), lambda i: (0, i, 0)),
            pl.BlockSpec((MB, 256), lambda i: (i, 0)),
            pl.BlockSpec((1, 256), lambda i: (0, 0)),
        ],
        out_specs=[
            pl.BlockSpec((MB, 256), lambda i: (i, 0)),
            pl.BlockSpec((2, 256), lambda i: (0, 0)),
        ],
        out_shape=[
            jax.ShapeDtypeStruct((N, 256), jnp.float32),
            jax.ShapeDtypeStruct((2, 256), jnp.float32),
        ],
    )(agg0, agg0, cnt_p, xroot0, b0r)

    h0b = pl.pallas_call(
        functools.partial(_stage_b_body, N),
        grid=(grid,),
        in_specs=[
            pl.BlockSpec((MB, 256), lambda i: (i, 0)),
            pl.BlockSpec((2, 256), lambda i: (0, 0)),
            pl.BlockSpec((1, 256), lambda i: (0, 0)),
            pl.BlockSpec((1, 256), lambda i: (0, 0)),
        ],
        out_specs=pl.BlockSpec((MB, 256), lambda i: (i, 0)),
        out_shape=jax.ShapeDtypeStruct((N, 256), jnp.bfloat16),
    )(hpre, stats, g0r, be0r)

    tbl1, hroot1 = _mm_call(h0b, wcat1, N, R, False)
    agg1 = sc_agg(tbl1.reshape(NC, N * R, 128), idx_all, zacc)

    out = pl.pallas_call(
        _stage_c_body,
        grid=(grid,),
        in_specs=[
            _half_spec(0),
            _half_spec(1),
            pl.BlockSpec((NC, MB, 128), lambda i: (0, i, 0)),
            pl.BlockSpec((MB, 256), lambda i: (i, 0)),
            pl.BlockSpec((1, 256), lambda i: (0, 0)),
            pl.BlockSpec((MB, 256), lambda i: (i, 0)),
            pl.BlockSpec((1, 256), lambda i: (0, 0)),
        ],
        out_specs=pl.BlockSpec((MB, 256), lambda i: (i, 0)),
        out_shape=jax.ShapeDtypeStruct((N, 256), jnp.float32),
    )(agg1, agg1, cnt_p, hroot1, b1r, xskip, sbr)
    return out


# speedup vs baseline: 1.2306x; 1.2177x over previous
import functools

import jax
import jax.numpy as jnp
from jax import lax
from jax.experimental import pallas as pl
from jax.experimental.pallas import tpu as pltpu
from jax.experimental.pallas import tpu_sc as plsc

EPS = 1e-5

NC = 2
NS = 16
CH = 128
IB = 4
MB = 400


def _elu(v):
    return jnp.where(v > 0, v, jnp.exp(jnp.minimum(v, 0.0)) - 1.0)


def _mm_body(has_skip, RH, x_ref, w_ref, tbl_ref, root_ref, *rest):
    acc = jnp.dot(x_ref[...], w_ref[...], preferred_element_type=jnp.float32)
    tbl_ref[0] = acc[:, :RH]
    tbl_ref[1] = acc[:, RH:2 * RH]
    root_ref[...] = acc[:, 2 * RH:2 * RH + 256]
    if has_skip:
        rest[0][...] = acc[:, 2 * RH + 256:2 * RH + 512]


def _mm_call(xb, wcat, N, R, has_skip):
    RH = R * 128
    KW = wcat.shape[1]
    grid = N // MB
    outs = [
        jax.ShapeDtypeStruct((NC, N, RH), jnp.float32),
        jax.ShapeDtypeStruct((N, 256), jnp.float32),
    ]
    out_specs = [
        pl.BlockSpec((NC, MB, RH), lambda i: (0, i, 0)),
        pl.BlockSpec((MB, 256), lambda i: (i, 0)),
    ]
    if has_skip:
        outs.append(jax.ShapeDtypeStruct((N, 256), jnp.float32))
        out_specs.append(pl.BlockSpec((MB, 256), lambda i: (i, 0)))
    return pl.pallas_call(
        functools.partial(_mm_body, has_skip, RH),
        grid=(grid,),
        in_specs=[
            pl.BlockSpec((MB, xb.shape[1]), lambda i: (i, 0)),
            pl.BlockSpec((xb.shape[1], KW), lambda i: (0, 0)),
        ],
        out_specs=out_specs,
        out_shape=outs,
    )(xb, wcat)


def _make_sc_agg(N, R, ACC, ETP):
    NCH = ETP // CH
    rows_per = ACC // NS
    mesh = plsc.VectorSubcoreMesh(core_axis_name="c", subcore_axis_name="s")

    def body(table, idx_hbm, zacc, out, acc_sh, ibuf, buf0, buf1, sem0, sem1):
        cid = lax.axis_index("c")
        sid = lax.axis_index("s")
        r0 = sid * rows_per
        pltpu.sync_copy(zacc.at[pl.ds(r0, rows_per)],
                        acc_sh.at[pl.ds(r0, rows_per)])
        plsc.subcore_barrier()

        def step(t, carry):
            pltpu.sync_copy(idx_hbm.at[sid].at[pl.ds(2 * t, 2)], ibuf)
            dA = pltpu.async_copy(
                table.at[cid].at[ibuf.at[0].at[0]], buf0, sem0)
            dB = pltpu.async_copy(
                table.at[cid].at[ibuf.at[1].at[0]], buf1, sem1)
            dA.wait()
            pltpu.sync_copy(buf0, acc_sh.at[ibuf.at[0].at[1]], add=True)
            dB.wait()
            pltpu.sync_copy(buf1, acc_sh.at[ibuf.at[1].at[1]], add=True)
            return carry

        lax.fori_loop(0, NCH // 2, step, 0)
        plsc.subcore_barrier()
        pltpu.sync_copy(acc_sh.at[pl.ds(r0, rows_per)],
                        out.at[cid].at[pl.ds(r0, rows_per)])

    scratch = [
        pltpu.VMEM_SHARED((ACC, 128), jnp.float32),
        pltpu.VMEM((2, 2, CH), jnp.int32),
        pltpu.VMEM((CH, 128), jnp.float32),
        pltpu.VMEM((CH, 128), jnp.float32),
        pltpu.SemaphoreType.DMA,
        pltpu.SemaphoreType.DMA,
    ]
    return pl.kernel(body, out_type=jax.ShapeDtypeStruct((NC, ACC, 128),
                                                         jnp.float32),
                     mesh=mesh, scratch_types=scratch)


def _make_sc_cnt(ACC, ETC):
    NCHC = ETC // CH
    rows_per = ACC // NS
    mesh = plsc.VectorSubcoreMesh(core_axis_name="c", subcore_axis_name="s")

    def body(didx, zcnt, ones, cnt_out, cnt_sh, ibuf, ones_v):
        cid = lax.axis_index("c")
        sid = lax.axis_index("s")
        wid = cid * NS + sid
        r0 = sid * rows_per
        pltpu.sync_copy(zcnt.at[pl.ds(r0, rows_per)],
                        cnt_sh.at[pl.ds(r0, rows_per)])
        pltpu.sync_copy(ones, ones_v)
        plsc.subcore_barrier()

        def step(t, carry):
            pltpu.sync_copy(didx.at[wid].at[pl.ds(2 * t, 2)], ibuf)
            pltpu.sync_copy(ones_v, cnt_sh.at[ibuf.at[0].at[0]], add=True)
            pltpu.sync_copy(ones_v, cnt_sh.at[ibuf.at[1].at[0]], add=True)
            return carry

        lax.fori_loop(0, NCHC // 2, step, 0)
        plsc.subcore_barrier()
        pltpu.sync_copy(cnt_sh.at[pl.ds(r0, rows_per)],
                        cnt_out.at[cid].at[pl.ds(r0, rows_per)])

    scratch = [
        pltpu.VMEM_SHARED((ACC, CH), jnp.float32),
        pltpu.VMEM((2, 1, CH), jnp.int32),
        pltpu.VMEM((CH, CH), jnp.float32),
    ]
    return pl.kernel(body, out_type=jax.ShapeDtypeStruct((NC, ACC, CH),
                                                         jnp.float32),
                     mesh=mesh, scratch_types=scratch)


def _stage_a_body(N, a0_ref, a1_ref, cnt_ref, root_ref, b_ref,
                  hpre_ref, stats_ref):
    i = pl.program_id(0)
    cnt = jnp.maximum(jnp.sum(cnt_ref[...], axis=(0, 2)) * (1.0 / 128.0), 1.0)
    h = (jnp.concatenate([a0_ref[0], a1_ref[0]], axis=1) / cnt[:, None]
         + root_ref[...] + b_ref[...])
    hpre_ref[...] = h.astype(jnp.bfloat16)
    s = jnp.concatenate([jnp.sum(h, axis=0, keepdims=True),
                         jnp.sum(h * h, axis=0, keepdims=True)], axis=0)

    @pl.when(i == 0)
    def _():
        stats_ref[...] = s

    @pl.when(i > 0)
    def _():
        stats_ref[...] += s


def _mm1_body(N, RH, hpre_ref, stats_ref, g_ref, be_ref, w_ref,
              tbl_ref, root_ref):
    s = stats_ref[...]
    mean = s[0:1] * (1.0 / N)
    var = s[1:2] * (1.0 / N) - mean * mean
    inv = lax.rsqrt(var + EPS) * g_ref[...]
    y = (hpre_ref[...].astype(jnp.float32) - mean) * inv + be_ref[...]
    xb = _elu(y).astype(jnp.bfloat16)
    acc = jnp.dot(xb, w_ref[...], preferred_element_type=jnp.float32)
    tbl_ref[0] = acc[:, :RH]
    tbl_ref[1] = acc[:, RH:2 * RH]
    root_ref[...] = acc[:, 2 * RH:2 * RH + 256]


def _mm1_call(hpre, stats, g0r, be0r, wcat, N, R):
    RH = R * 128
    KW = wcat.shape[1]
    return pl.pallas_call(
        functools.partial(_mm1_body, N, RH),
        grid=(N // MB,),
        in_specs=[
            pl.BlockSpec((MB, 256), lambda i: (i, 0)),
            pl.BlockSpec((2, 256), lambda i: (0, 0)),
            pl.BlockSpec((1, 256), lambda i: (0, 0)),
            pl.BlockSpec((1, 256), lambda i: (0, 0)),
            pl.BlockSpec((256, KW), lambda i: (0, 0)),
        ],
        out_specs=[
            pl.BlockSpec((NC, MB, RH), lambda i: (0, i, 0)),
            pl.BlockSpec((MB, 256), lambda i: (i, 0)),
        ],
        out_shape=[
            jax.ShapeDtypeStruct((NC, N, RH), jnp.float32),
            jax.ShapeDtypeStruct((N, 256), jnp.float32),
        ],
    )(hpre, stats, g0r, be0r, wcat)


def _stage_c_body(a0_ref, a1_ref, cnt_ref, root_ref, b_ref, skip_ref,
                  sb_ref, out_ref):
    cnt = jnp.maximum(jnp.sum(cnt_ref[...], axis=(0, 2)) * (1.0 / 128.0), 1.0)
    h = (jnp.concatenate([a0_ref[0], a1_ref[0]], axis=1) / cnt[:, None]
         + root_ref[...] + b_ref[...])
    h = _elu(h)
    h = h + skip_ref[...] + sb_ref[...]
    out_ref[...] = _elu(h)


def _half_spec(c):
    return pl.BlockSpec((1, MB, 128), lambda i, c=c: (c, i, 0))


def kernel(x, edge_index, edge_types, w0, root0, b0, w1, root1, b1,
           gamma0, beta0, skip_w, skip_b):
    N, F = x.shape
    R = w0.shape[0]
    E = edge_index.shape[1]

    NW = NC * NS
    ETP = -(-E // NS // (2 * IB * CH)) * (2 * IB * CH)
    EP = NS * ETP
    ACC = -(-(N + 1) // (NS * 8)) * (NS * 8)
    NCH = ETP // CH

    src = edge_index[0].astype(jnp.int32)
    dst = edge_index[1].astype(jnp.int32)
    ety = edge_types.astype(jnp.int32)
    pad = EP - E
    src_p = jnp.concatenate([src, jnp.zeros((pad,), jnp.int32)])
    ety_p = jnp.concatenate([ety, jnp.zeros((pad,), jnp.int32)])
    dst_p = jnp.concatenate([dst, jnp.full((pad,), N, jnp.int32)])
    gidx = (src_p * R + ety_p).reshape(NS, NCH, CH)
    dst2 = dst_p.reshape(NS, NCH, CH)
    idx_all = jnp.stack([gidx, dst2], axis=2)
    ETC = EP // NW
    didx = dst_p.reshape(NW, ETC // CH, 1, CH)
    zacc = jnp.zeros((ACC, 128), jnp.float32)
    zcnt = jnp.zeros((ACC, CH), jnp.float32)
    ones = jnp.ones((CH, CH), jnp.float32)

    def wcat_of(w, extra):
        h0 = w[:, :, :128].transpose(1, 0, 2).reshape(F, R * 128)
        h1 = w[:, :, 128:].transpose(1, 0, 2).reshape(F, R * 128)
        return jnp.concatenate([h0, h1] + extra, axis=1).astype(jnp.bfloat16)

    wcat0 = wcat_of(w0, [root0, skip_w])
    wcat1 = wcat_of(w1, [root1])
    xb = x.astype(jnp.bfloat16)
    b0r = b0.reshape(1, 256)
    b1r = b1.reshape(1, 256)
    g0r = gamma0.reshape(1, 256)
    be0r = beta0.reshape(1, 256)
    sbr = skip_b.reshape(1, 256)

    sc_agg = _make_sc_agg(N, R, ACC, ETP)
    sc_cnt = _make_sc_cnt(ACC, ETC)

    cnt_p = sc_cnt(didx, zcnt, ones)
    tbl0, xroot0, xskip = _mm_call(xb, wcat0, N, R, True)
    agg0 = sc_agg(tbl0.reshape(NC, N * R, 128), idx_all, zacc)

    grid = N // MB
    hpre, stats = pl.pallas_call(
        functools.partial(_stage_a_body, N),
        grid=(grid,),
        in_specs=[
            _half_spec(0),
            _half_spec(1),
            pl.BlockSpec((NC, MB, 128), lambda i: (0, i, 0)),
            pl.BlockSpec((MB, 256), lambda i: (i, 0)),
            pl.BlockSpec((1, 256), lambda i: (0, 0)),
        ],
        out_specs=[
            pl.BlockSpec((MB, 256), lambda i: (i, 0)),
            pl.BlockSpec((2, 256), lambda i: (0, 0)),
        ],
        out_shape=[
            jax.ShapeDtypeStruct((N, 256), jnp.bfloat16),
            jax.ShapeDtypeStruct((2, 256), jnp.float32),
        ],
    )(agg0, agg0, cnt_p, xroot0, b0r)

    tbl1, hroot1 = _mm1_call(hpre, stats, g0r, be0r, wcat1, N, R)
    agg1 = sc_agg(tbl1.reshape(NC, N * R, 128), idx_all, zacc)

    out = pl.pallas_call(
        _stage_c_body,
        grid=(grid,),
        in_specs=[
            _half_spec(0),
            _half_spec(1),
            pl.BlockSpec((NC, MB, 128), lambda i: (0, i, 0)),
            pl.BlockSpec((MB, 256), lambda i: (i, 0)),
            pl.BlockSpec((1, 256), lambda i: (0, 0)),
            pl.BlockSpec((MB, 256), lambda i: (i, 0)),
            pl.BlockSpec((1, 256), lambda i: (0, 0)),
        ],
        out_specs=pl.BlockSpec((MB, 256), lambda i: (i, 0)),
        out_shape=jax.ShapeDtypeStruct((N, 256), jnp.float32),
    )(agg1, agg1, cnt_p, hroot1, b1r, xskip, sbr)
    return out


# speedup vs baseline: 1.3060x; 1.0612x over previous
import functools

import jax
import jax.numpy as jnp
from jax import lax
from jax.experimental import pallas as pl
from jax.experimental.pallas import tpu as pltpu
from jax.experimental.pallas import tpu_sc as plsc

EPS = 1e-5

NC = 2
NS = 16
CH = 128
IB = 4
MB = 400


def _elu(v):
    return jnp.where(v > 0, v, jnp.exp(jnp.minimum(v, 0.0)) - 1.0)


def _mm_body(has_skip, RH, x_ref, w_ref, tbl_ref, root_ref, *rest):
    acc = jnp.dot(x_ref[...], w_ref[...], preferred_element_type=jnp.float32)
    tbl_ref[0] = acc[:, :RH]
    tbl_ref[1] = acc[:, RH:2 * RH]
    root_ref[...] = acc[:, 2 * RH:2 * RH + 256]
    if has_skip:
        rest[0][...] = acc[:, 2 * RH + 256:2 * RH + 512]


def _mm_call(xb, wcat, N, R, has_skip):
    RH = R * 128
    KW = wcat.shape[1]
    grid = N // MB
    outs = [
        jax.ShapeDtypeStruct((NC, N, RH), jnp.float32),
        jax.ShapeDtypeStruct((N, 256), jnp.float32),
    ]
    out_specs = [
        pl.BlockSpec((NC, MB, RH), lambda i: (0, i, 0)),
        pl.BlockSpec((MB, 256), lambda i: (i, 0)),
    ]
    if has_skip:
        outs.append(jax.ShapeDtypeStruct((N, 256), jnp.float32))
        out_specs.append(pl.BlockSpec((MB, 256), lambda i: (i, 0)))
    return pl.pallas_call(
        functools.partial(_mm_body, has_skip, RH),
        grid=(grid,),
        in_specs=[
            pl.BlockSpec((MB, xb.shape[1]), lambda i: (i, 0)),
            pl.BlockSpec((xb.shape[1], KW), lambda i: (0, 0)),
        ],
        out_specs=out_specs,
        out_shape=outs,
    )(xb, wcat)


def _make_sc_agg(N, R, ACC, ETP):
    NCH = ETP // CH
    rows_per = ACC // NS
    mesh = plsc.VectorSubcoreMesh(core_axis_name="c", subcore_axis_name="s")

    def body(table, idx_hbm, zacc, out, acc_sh, ibuf, buf0, buf1, sem0, sem1):
        cid = lax.axis_index("c")
        sid = lax.axis_index("s")
        r0 = sid * rows_per
        pltpu.sync_copy(zacc.at[pl.ds(r0, rows_per)],
                        acc_sh.at[pl.ds(r0, rows_per)])
        plsc.subcore_barrier()

        def step(t, carry):
            pltpu.sync_copy(idx_hbm.at[sid].at[pl.ds(4 * t, 4)], ibuf)
            d0 = pltpu.async_copy(
                table.at[cid].at[ibuf.at[0].at[0]], buf0, sem0)
            d1 = pltpu.async_copy(
                table.at[cid].at[ibuf.at[1].at[0]], buf1, sem1)
            d0.wait()
            pltpu.sync_copy(buf0, acc_sh.at[ibuf.at[0].at[1]], add=True)
            d2 = pltpu.async_copy(
                table.at[cid].at[ibuf.at[2].at[0]], buf0, sem0)
            d1.wait()
            pltpu.sync_copy(buf1, acc_sh.at[ibuf.at[1].at[1]], add=True)
            d3 = pltpu.async_copy(
                table.at[cid].at[ibuf.at[3].at[0]], buf1, sem1)
            d2.wait()
            pltpu.sync_copy(buf0, acc_sh.at[ibuf.at[2].at[1]], add=True)
            d3.wait()
            pltpu.sync_copy(buf1, acc_sh.at[ibuf.at[3].at[1]], add=True)
            return carry

        lax.fori_loop(0, NCH // 4, step, 0)
        plsc.subcore_barrier()
        pltpu.sync_copy(acc_sh.at[pl.ds(r0, rows_per)],
                        out.at[cid].at[pl.ds(r0, rows_per)])

    scratch = [
        pltpu.VMEM_SHARED((ACC, 128), jnp.float32),
        pltpu.VMEM((4, 2, CH), jnp.int32),
        pltpu.VMEM((CH, 128), jnp.float32),
        pltpu.VMEM((CH, 128), jnp.float32),
        pltpu.SemaphoreType.DMA,
        pltpu.SemaphoreType.DMA,
    ]
    return pl.kernel(body, out_type=jax.ShapeDtypeStruct((NC, ACC, 128),
                                                         jnp.float32),
                     mesh=mesh, scratch_types=scratch)


def _make_sc_cnt(ACC, ETC):
    NCHC = ETC // CH
    rows_per = ACC // NS
    mesh = plsc.VectorSubcoreMesh(core_axis_name="c", subcore_axis_name="s")

    def body(didx, zcnt, ones, cnt_out, cnt_sh, ibuf, ones_v):
        cid = lax.axis_index("c")
        sid = lax.axis_index("s")
        wid = cid * NS + sid
        r0 = sid * rows_per
        pltpu.sync_copy(zcnt.at[pl.ds(r0, rows_per)],
                        cnt_sh.at[pl.ds(r0, rows_per)])
        pltpu.sync_copy(ones, ones_v)
        plsc.subcore_barrier()

        def step(t, carry):
            pltpu.sync_copy(didx.at[wid].at[pl.ds(4 * t, 4)], ibuf)
            for q in range(4):
                pltpu.sync_copy(ones_v, cnt_sh.at[ibuf.at[q].at[0]],
                                add=True)
            return carry

        lax.fori_loop(0, NCHC // 4, step, 0)
        plsc.subcore_barrier()
        pltpu.sync_copy(cnt_sh.at[pl.ds(r0, rows_per)],
                        cnt_out.at[cid].at[pl.ds(r0, rows_per)])

    scratch = [
        pltpu.VMEM_SHARED((ACC, CH), jnp.float32),
        pltpu.VMEM((4, 1, CH), jnp.int32),
        pltpu.VMEM((CH, CH), jnp.float32),
    ]
    return pl.kernel(body, out_type=jax.ShapeDtypeStruct((NC, ACC, CH),
                                                         jnp.float32),
                     mesh=mesh, scratch_types=scratch)


def _stage_a_body(N, a0_ref, a1_ref, cnt_ref, root_ref, b_ref,
                  hpre_ref, stats_ref):
    i = pl.program_id(0)
    cnt = jnp.maximum(jnp.sum(cnt_ref[...], axis=(0, 2)) * (1.0 / 128.0), 1.0)
    h = (jnp.concatenate([a0_ref[0], a1_ref[0]], axis=1) / cnt[:, None]
         + root_ref[...] + b_ref[...])
    hpre_ref[...] = h.astype(jnp.bfloat16)
    s = jnp.concatenate([jnp.sum(h, axis=0, keepdims=True),
                         jnp.sum(h * h, axis=0, keepdims=True)], axis=0)

    @pl.when(i == 0)
    def _():
        stats_ref[...] = s

    @pl.when(i > 0)
    def _():
        stats_ref[...] += s


def _mm1_body(N, RH, hpre_ref, stats_ref, g_ref, be_ref, w_ref,
              tbl_ref, root_ref):
    s = stats_ref[...]
    mean = s[0:1] * (1.0 / N)
    var = s[1:2] * (1.0 / N) - mean * mean
    inv = lax.rsqrt(var + EPS) * g_ref[...]
    y = (hpre_ref[...].astype(jnp.float32) - mean) * inv + be_ref[...]
    xb = _elu(y).astype(jnp.bfloat16)
    acc = jnp.dot(xb, w_ref[...], preferred_element_type=jnp.float32)
    tbl_ref[0] = acc[:, :RH]
    tbl_ref[1] = acc[:, RH:2 * RH]
    root_ref[...] = acc[:, 2 * RH:2 * RH + 256]


def _mm1_call(hpre, stats, g0r, be0r, wcat, N, R):
    RH = R * 128
    KW = wcat.shape[1]
    return pl.pallas_call(
        functools.partial(_mm1_body, N, RH),
        grid=(N // MB,),
        in_specs=[
            pl.BlockSpec((MB, 256), lambda i: (i, 0)),
            pl.BlockSpec((2, 256), lambda i: (0, 0)),
            pl.BlockSpec((1, 256), lambda i: (0, 0)),
            pl.BlockSpec((1, 256), lambda i: (0, 0)),
            pl.BlockSpec((256, KW), lambda i: (0, 0)),
        ],
        out_specs=[
            pl.BlockSpec((NC, MB, RH), lambda i: (0, i, 0)),
            pl.BlockSpec((MB, 256), lambda i: (i, 0)),
        ],
        out_shape=[
            jax.ShapeDtypeStruct((NC, N, RH), jnp.float32),
            jax.ShapeDtypeStruct((N, 256), jnp.float32),
        ],
    )(hpre, stats, g0r, be0r, wcat)


def _stage_c_body(a0_ref, a1_ref, cnt_ref, root_ref, b_ref, skip_ref,
                  sb_ref, out_ref):
    cnt = jnp.maximum(jnp.sum(cnt_ref[...], axis=(0, 2)) * (1.0 / 128.0), 1.0)
    h = (jnp.concatenate([a0_ref[0], a1_ref[0]], axis=1) / cnt[:, None]
         + root_ref[...] + b_ref[...])
    h = _elu(h)
    h = h + skip_ref[...] + sb_ref[...]
    out_ref[...] = _elu(h)


def _half_spec(c):
    return pl.BlockSpec((1, MB, 128), lambda i, c=c: (c, i, 0))


def kernel(x, edge_index, edge_types, w0, root0, b0, w1, root1, b1,
           gamma0, beta0, skip_w, skip_b):
    N, F = x.shape
    R = w0.shape[0]
    E = edge_index.shape[1]

    NW = NC * NS
    ETP = -(-E // NS // (2 * IB * CH)) * (2 * IB * CH)
    EP = NS * ETP
    ACC = -(-(N + 1) // (NS * 8)) * (NS * 8)
    NCH = ETP // CH

    src = edge_index[0].astype(jnp.int32)
    dst = edge_index[1].astype(jnp.int32)
    ety = edge_types.astype(jnp.int32)
    pad = EP - E
    src_p = jnp.concatenate([src, jnp.zeros((pad,), jnp.int32)])
    ety_p = jnp.concatenate([ety, jnp.zeros((pad,), jnp.int32)])
    dst_p = jnp.concatenate([dst, jnp.full((pad,), N, jnp.int32)])
    gidx = (src_p * R + ety_p).reshape(NS, NCH, CH)
    dst2 = dst_p.reshape(NS, NCH, CH)
    idx_all = jnp.stack([gidx, dst2], axis=2)
    ETC = EP // NW
    didx = dst_p.reshape(NW, ETC // CH, 1, CH)
    zacc = jnp.zeros((ACC, 128), jnp.float32)
    zcnt = jnp.zeros((ACC, CH), jnp.float32)
    ones = jnp.ones((CH, CH), jnp.float32)

    def wcat_of(w, extra):
        h0 = w[:, :, :128].transpose(1, 0, 2).reshape(F, R * 128)
        h1 = w[:, :, 128:].transpose(1, 0, 2).reshape(F, R * 128)
        return jnp.concatenate([h0, h1] + extra, axis=1).astype(jnp.bfloat16)

    wcat0 = wcat_of(w0, [root0, skip_w])
    wcat1 = wcat_of(w1, [root1])
    xb = x.astype(jnp.bfloat16)
    b0r = b0.reshape(1, 256)
    b1r = b1.reshape(1, 256)
    g0r = gamma0.reshape(1, 256)
    be0r = beta0.reshape(1, 256)
    sbr = skip_b.reshape(1, 256)

    sc_agg = _make_sc_agg(N, R, ACC, ETP)
    sc_cnt = _make_sc_cnt(ACC, ETC)

    cnt_p = sc_cnt(didx, zcnt, ones)
    tbl0, xroot0, xskip = _mm_call(xb, wcat0, N, R, True)
    agg0 = sc_agg(tbl0.reshape(NC, N * R, 128), idx_all, zacc)

    grid = N // MB
    hpre, stats = pl.pallas_call(
        functools.partial(_stage_a_body, N),
        grid=(grid,),
        in_specs=[
            _half_spec(0),
            _half_spec(1),
            pl.BlockSpec((NC, MB, 128), lambda i: (0, i, 0)),
            pl.BlockSpec((MB, 256), lambda i: (i, 0)),
            pl.BlockSpec((1, 256), lambda i: (0, 0)),
        ],
        out_specs=[
            pl.BlockSpec((MB, 256), lambda i: (i, 0)),
            pl.BlockSpec((2, 256), lambda i: (0, 0)),
        ],
        out_shape=[
            jax.ShapeDtypeStruct((N, 256), jnp.bfloat16),
            jax.ShapeDtypeStruct((2, 256), jnp.float32),
        ],
    )(agg0, agg0, cnt_p, xroot0, b0r)

    tbl1, hroot1 = _mm1_call(hpre, stats, g0r, be0r, wcat1, N, R)
    agg1 = sc_agg(tbl1.reshape(NC, N * R, 128), idx_all, zacc)

    out = pl.pallas_call(
        _stage_c_body,
        grid=(grid,),
        in_specs=[
            _half_spec(0),
            _half_spec(1),
            pl.BlockSpec((NC, MB, 128), lambda i: (0, i, 0)),
            pl.BlockSpec((MB, 256), lambda i: (i, 0)),
            pl.BlockSpec((1, 256), lambda i: (0, 0)),
            pl.BlockSpec((MB, 256), lambda i: (i, 0)),
            pl.BlockSpec((1, 256), lambda i: (0, 0)),
        ],
        out_specs=pl.BlockSpec((MB, 256), lambda i: (i, 0)),
        out_shape=jax.ShapeDtypeStruct((N, 256), jnp.float32),
    )(agg1, agg1, cnt_p, hroot1, b1r, xskip, sbr)
    return out


# speedup vs baseline: 1.3542x; 1.0369x over previous
import functools

import jax
import jax.numpy as jnp
from jax import lax
from jax.experimental import pallas as pl
from jax.experimental.pallas import tpu as pltpu
from jax.experimental.pallas import tpu_sc as plsc

EPS = 1e-5

NC = 2
NS = 16
CH = 128
IB = 4
MB = 400


def _elu(v):
    return jnp.where(v > 0, v, jnp.exp(jnp.minimum(v, 0.0)) - 1.0)


def _mm_body(has_skip, RH, x_ref, w_ref, tbl_ref, root_ref, *rest):
    acc = jnp.dot(x_ref[...], w_ref[...], preferred_element_type=jnp.float32)
    tbl_ref[0] = acc[:, :RH]
    tbl_ref[1] = acc[:, RH:2 * RH]
    root_ref[...] = acc[:, 2 * RH:2 * RH + 256]
    if has_skip:
        rest[0][...] = acc[:, 2 * RH + 256:2 * RH + 512]


def _mm_call(xb, wcat, N, R, has_skip):
    RH = R * 128
    KW = wcat.shape[1]
    grid = N // MB
    outs = [
        jax.ShapeDtypeStruct((NC, N, RH), jnp.float32),
        jax.ShapeDtypeStruct((N, 256), jnp.float32),
    ]
    out_specs = [
        pl.BlockSpec((NC, MB, RH), lambda i: (0, i, 0)),
        pl.BlockSpec((MB, 256), lambda i: (i, 0)),
    ]
    if has_skip:
        outs.append(jax.ShapeDtypeStruct((N, 256), jnp.float32))
        out_specs.append(pl.BlockSpec((MB, 256), lambda i: (i, 0)))
    return pl.pallas_call(
        functools.partial(_mm_body, has_skip, RH),
        grid=(grid,),
        in_specs=[
            pl.BlockSpec((MB, xb.shape[1]), lambda i: (i, 0)),
            pl.BlockSpec((xb.shape[1], KW), lambda i: (0, 0)),
        ],
        out_specs=out_specs,
        out_shape=outs,
    )(xb, wcat)


def _make_sc_agg(N, R, ACC, ETP):
    NCH = ETP // CH
    rows_per = ACC // NS
    mesh = plsc.VectorSubcoreMesh(core_axis_name="c", subcore_axis_name="s")

    def body(table, idx_hbm, zacc, out, acc_sh, ibuf, buf0, buf1, sem0, sem1):
        cid = lax.axis_index("c")
        sid = lax.axis_index("s")
        r0 = sid * rows_per
        pltpu.sync_copy(zacc.at[pl.ds(r0, rows_per)],
                        acc_sh.at[pl.ds(r0, rows_per)])
        plsc.subcore_barrier()

        QI = 8

        def step(t, carry):
            pltpu.sync_copy(idx_hbm.at[sid].at[pl.ds(QI * t, QI)], ibuf)
            bufs = [buf0, buf1]
            sems = [sem0, sem1]
            pend = [
                pltpu.async_copy(table.at[cid].at[ibuf.at[0].at[0]],
                                 buf0, sem0),
                pltpu.async_copy(table.at[cid].at[ibuf.at[1].at[0]],
                                 buf1, sem1),
            ]
            for q in range(QI):
                p = q % 2
                pend[p].wait()
                pltpu.sync_copy(bufs[p], acc_sh.at[ibuf.at[q].at[1]],
                                add=True)
                if q + 2 < QI:
                    pend[p] = pltpu.async_copy(
                        table.at[cid].at[ibuf.at[q + 2].at[0]],
                        bufs[p], sems[p])
            return carry

        lax.fori_loop(0, NCH // QI, step, 0)
        plsc.subcore_barrier()
        pltpu.sync_copy(acc_sh.at[pl.ds(r0, rows_per)],
                        out.at[cid].at[pl.ds(r0, rows_per)])

    scratch = [
        pltpu.VMEM_SHARED((ACC, 128), jnp.float32),
        pltpu.VMEM((8, 2, CH), jnp.int32),
        pltpu.VMEM((CH, 128), jnp.float32),
        pltpu.VMEM((CH, 128), jnp.float32),
        pltpu.SemaphoreType.DMA,
        pltpu.SemaphoreType.DMA,
    ]
    return pl.kernel(body, out_type=jax.ShapeDtypeStruct((NC, ACC, 128),
                                                         jnp.float32),
                     mesh=mesh, scratch_types=scratch)


def _make_sc_cnt(ACC, ETC):
    NCHC = ETC // CH
    rows_per = ACC // NS
    mesh = plsc.VectorSubcoreMesh(core_axis_name="c", subcore_axis_name="s")

    def body(didx, zcnt, ones, cnt_out, cnt_sh, ibuf, ones_v):
        cid = lax.axis_index("c")
        sid = lax.axis_index("s")
        wid = cid * NS + sid
        r0 = sid * rows_per
        pltpu.sync_copy(zcnt.at[pl.ds(r0, rows_per)],
                        cnt_sh.at[pl.ds(r0, rows_per)])
        pltpu.sync_copy(ones, ones_v)
        plsc.subcore_barrier()

        def step(t, carry):
            pltpu.sync_copy(didx.at[wid].at[pl.ds(4 * t, 4)], ibuf)
            for q in range(4):
                pltpu.sync_copy(ones_v, cnt_sh.at[ibuf.at[q].at[0]],
                                add=True)
            return carry

        lax.fori_loop(0, NCHC // 4, step, 0)
        plsc.subcore_barrier()
        pltpu.sync_copy(cnt_sh.at[pl.ds(r0, rows_per)],
                        cnt_out.at[cid].at[pl.ds(r0, rows_per)])

    scratch = [
        pltpu.VMEM_SHARED((ACC, CH), jnp.float32),
        pltpu.VMEM((4, 1, CH), jnp.int32),
        pltpu.VMEM((CH, CH), jnp.float32),
    ]
    return pl.kernel(body, out_type=jax.ShapeDtypeStruct((NC, ACC, CH),
                                                         jnp.float32),
                     mesh=mesh, scratch_types=scratch)


def _stage_a_body(N, a0_ref, a1_ref, cnt_ref, root_ref, b_ref,
                  hpre_ref, stats_ref):
    i = pl.program_id(0)
    cnt = jnp.maximum(jnp.sum(cnt_ref[...], axis=(0, 2)) * (1.0 / 128.0), 1.0)
    h = (jnp.concatenate([a0_ref[0], a1_ref[0]], axis=1) / cnt[:, None]
         + root_ref[...] + b_ref[...])
    hpre_ref[...] = h.astype(jnp.bfloat16)
    s = jnp.concatenate([jnp.sum(h, axis=0, keepdims=True),
                         jnp.sum(h * h, axis=0, keepdims=True)], axis=0)

    @pl.when(i == 0)
    def _():
        stats_ref[...] = s

    @pl.when(i > 0)
    def _():
        stats_ref[...] += s


def _mm1_body(N, RH, hpre_ref, stats_ref, g_ref, be_ref, w_ref,
              tbl_ref, root_ref):
    s = stats_ref[...]
    mean = s[0:1] * (1.0 / N)
    var = s[1:2] * (1.0 / N) - mean * mean
    inv = lax.rsqrt(var + EPS) * g_ref[...]
    y = (hpre_ref[...].astype(jnp.float32) - mean) * inv + be_ref[...]
    xb = _elu(y).astype(jnp.bfloat16)
    acc = jnp.dot(xb, w_ref[...], preferred_element_type=jnp.float32)
    tbl_ref[0] = acc[:, :RH]
    tbl_ref[1] = acc[:, RH:2 * RH]
    root_ref[...] = acc[:, 2 * RH:2 * RH + 256]


def _mm1_call(hpre, stats, g0r, be0r, wcat, N, R):
    RH = R * 128
    KW = wcat.shape[1]
    return pl.pallas_call(
        functools.partial(_mm1_body, N, RH),
        grid=(N // MB,),
        in_specs=[
            pl.BlockSpec((MB, 256), lambda i: (i, 0)),
            pl.BlockSpec((2, 256), lambda i: (0, 0)),
            pl.BlockSpec((1, 256), lambda i: (0, 0)),
            pl.BlockSpec((1, 256), lambda i: (0, 0)),
            pl.BlockSpec((256, KW), lambda i: (0, 0)),
        ],
        out_specs=[
            pl.BlockSpec((NC, MB, RH), lambda i: (0, i, 0)),
            pl.BlockSpec((MB, 256), lambda i: (i, 0)),
        ],
        out_shape=[
            jax.ShapeDtypeStruct((NC, N, RH), jnp.float32),
            jax.ShapeDtypeStruct((N, 256), jnp.float32),
        ],
    )(hpre, stats, g0r, be0r, wcat)


def _stage_c_body(a0_ref, a1_ref, cnt_ref, root_ref, b_ref, skip_ref,
                  sb_ref, out_ref):
    cnt = jnp.maximum(jnp.sum(cnt_ref[...], axis=(0, 2)) * (1.0 / 128.0), 1.0)
    h = (jnp.concatenate([a0_ref[0], a1_ref[0]], axis=1) / cnt[:, None]
         + root_ref[...] + b_ref[...])
    h = _elu(h)
    h = h + skip_ref[...] + sb_ref[...]
    out_ref[...] = _elu(h)


def _half_spec(c):
    return pl.BlockSpec((1, MB, 128), lambda i, c=c: (c, i, 0))


def kernel(x, edge_index, edge_types, w0, root0, b0, w1, root1, b1,
           gamma0, beta0, skip_w, skip_b):
    N, F = x.shape
    R = w0.shape[0]
    E = edge_index.shape[1]

    NW = NC * NS
    ETP = -(-E // NS // (2 * IB * CH)) * (2 * IB * CH)
    EP = NS * ETP
    ACC = -(-(N + 1) // (NS * 8)) * (NS * 8)
    NCH = ETP // CH

    src = edge_index[0].astype(jnp.int32)
    dst = edge_index[1].astype(jnp.int32)
    ety = edge_types.astype(jnp.int32)
    pad = EP - E
    src_p = jnp.concatenate([src, jnp.zeros((pad,), jnp.int32)])
    ety_p = jnp.concatenate([ety, jnp.zeros((pad,), jnp.int32)])
    dst_p = jnp.concatenate([dst, jnp.full((pad,), N, jnp.int32)])
    gidx = (src_p * R + ety_p).reshape(NS, NCH, CH)
    dst2 = dst_p.reshape(NS, NCH, CH)
    idx_all = jnp.stack([gidx, dst2], axis=2)
    ETC = EP // NW
    didx = dst_p.reshape(NW, ETC // CH, 1, CH)
    zacc = jnp.zeros((ACC, 128), jnp.float32)
    zcnt = jnp.zeros((ACC, CH), jnp.float32)
    ones = jnp.ones((CH, CH), jnp.float32)

    def wcat_of(w, extra):
        h0 = w[:, :, :128].transpose(1, 0, 2).reshape(F, R * 128)
        h1 = w[:, :, 128:].transpose(1, 0, 2).reshape(F, R * 128)
        return jnp.concatenate([h0, h1] + extra, axis=1).astype(jnp.bfloat16)

    wcat0 = wcat_of(w0, [root0, skip_w])
    wcat1 = wcat_of(w1, [root1])
    xb = x.astype(jnp.bfloat16)
    b0r = b0.reshape(1, 256)
    b1r = b1.reshape(1, 256)
    g0r = gamma0.reshape(1, 256)
    be0r = beta0.reshape(1, 256)
    sbr = skip_b.reshape(1, 256)

    sc_agg = _make_sc_agg(N, R, ACC, ETP)
    sc_cnt = _make_sc_cnt(ACC, ETC)

    cnt_p = sc_cnt(didx, zcnt, ones)
    tbl0, xroot0, xskip = _mm_call(xb, wcat0, N, R, True)
    agg0 = sc_agg(tbl0.reshape(NC, N * R, 128), idx_all, zacc)

    grid = N // MB
    hpre, stats = pl.pallas_call(
        functools.partial(_stage_a_body, N),
        grid=(grid,),
        in_specs=[
            _half_spec(0),
            _half_spec(1),
            pl.BlockSpec((NC, MB, 128), lambda i: (0, i, 0)),
            pl.BlockSpec((MB, 256), lambda i: (i, 0)),
            pl.BlockSpec((1, 256), lambda i: (0, 0)),
        ],
        out_specs=[
            pl.BlockSpec((MB, 256), lambda i: (i, 0)),
            pl.BlockSpec((2, 256), lambda i: (0, 0)),
        ],
        out_shape=[
            jax.ShapeDtypeStruct((N, 256), jnp.bfloat16),
            jax.ShapeDtypeStruct((2, 256), jnp.float32),
        ],
    )(agg0, agg0, cnt_p, xroot0, b0r)

    tbl1, hroot1 = _mm1_call(hpre, stats, g0r, be0r, wcat1, N, R)
    agg1 = sc_agg(tbl1.reshape(NC, N * R, 128), idx_all, zacc)

    out = pl.pallas_call(
        _stage_c_body,
        grid=(grid,),
        in_specs=[
            _half_spec(0),
            _half_spec(1),
            pl.BlockSpec((NC, MB, 128), lambda i: (0, i, 0)),
            pl.BlockSpec((MB, 256), lambda i: (i, 0)),
            pl.BlockSpec((1, 256), lambda i: (0, 0)),
            pl.BlockSpec((MB, 256), lambda i: (i, 0)),
            pl.BlockSpec((1, 256), lambda i: (0, 0)),
        ],
        out_specs=pl.BlockSpec((MB, 256), lambda i: (i, 0)),
        out_shape=jax.ShapeDtypeStruct((N, 256), jnp.float32),
    )(agg1, agg1, cnt_p, hroot1, b1r, xskip, sbr)
    return out


# speedup vs baseline: 1.3572x; 1.0022x over previous
import functools

import jax
import jax.numpy as jnp
from jax import lax
from jax.experimental import pallas as pl
from jax.experimental.pallas import tpu as pltpu
from jax.experimental.pallas import tpu_sc as plsc

EPS = 1e-5

NC = 2
NS = 16
CH = 128
IB = 4
MB = 400


def _elu(v):
    return jnp.where(v > 0, v, jnp.exp(jnp.minimum(v, 0.0)) - 1.0)


def _mm_body(has_skip, RH, x_ref, w_ref, tbl_ref, root_ref, *rest):
    acc = jnp.dot(x_ref[...], w_ref[...], preferred_element_type=jnp.float32)
    tbl_ref[0] = acc[:, :RH]
    tbl_ref[1] = acc[:, RH:2 * RH]
    root_ref[...] = acc[:, 2 * RH:2 * RH + 256]
    if has_skip:
        rest[0][...] = acc[:, 2 * RH + 256:2 * RH + 512]


def _mm_call(xb, wcat, N, R, has_skip):
    RH = R * 128
    KW = wcat.shape[1]
    grid = N // MB
    outs = [
        jax.ShapeDtypeStruct((NC, N, RH), jnp.float32),
        jax.ShapeDtypeStruct((N, 256), jnp.float32),
    ]
    out_specs = [
        pl.BlockSpec((NC, MB, RH), lambda i: (0, i, 0)),
        pl.BlockSpec((MB, 256), lambda i: (i, 0)),
    ]
    if has_skip:
        outs.append(jax.ShapeDtypeStruct((N, 256), jnp.float32))
        out_specs.append(pl.BlockSpec((MB, 256), lambda i: (i, 0)))
    return pl.pallas_call(
        functools.partial(_mm_body, has_skip, RH),
        grid=(grid,),
        in_specs=[
            pl.BlockSpec((MB, xb.shape[1]), lambda i: (i, 0)),
            pl.BlockSpec((xb.shape[1], KW), lambda i: (0, 0)),
        ],
        out_specs=out_specs,
        out_shape=outs,
    )(xb, wcat)


def _make_sc_agg(N, R, ACC, ETP):
    NCH = ETP // CH
    rows_per = ACC // NS
    mesh = plsc.VectorSubcoreMesh(core_axis_name="c", subcore_axis_name="s")

    def body(table, idx_hbm, zacc, out, acc_sh, ibuf, buf0, buf1, sem0, sem1):
        cid = lax.axis_index("c")
        sid = lax.axis_index("s")
        r0 = sid * rows_per
        pltpu.sync_copy(zacc.at[pl.ds(r0, rows_per)],
                        acc_sh.at[pl.ds(r0, rows_per)])
        plsc.subcore_barrier()

        QI = 8

        def step(t, carry):
            pltpu.sync_copy(idx_hbm.at[sid].at[pl.ds(QI * t, QI)], ibuf)
            bufs = [buf0, buf1]
            sems = [sem0, sem1]
            pend = [
                pltpu.async_copy(table.at[cid].at[ibuf.at[0].at[0]],
                                 buf0, sem0),
                pltpu.async_copy(table.at[cid].at[ibuf.at[1].at[0]],
                                 buf1, sem1),
            ]
            for q in range(QI):
                p = q % 2
                pend[p].wait()
                pltpu.sync_copy(bufs[p], acc_sh.at[ibuf.at[q].at[1]],
                                add=True)
                if q + 2 < QI:
                    pend[p] = pltpu.async_copy(
                        table.at[cid].at[ibuf.at[q + 2].at[0]],
                        bufs[p], sems[p])
            return carry

        lax.fori_loop(0, NCH // QI, step, 0)
        plsc.subcore_barrier()
        pltpu.sync_copy(acc_sh.at[pl.ds(r0, rows_per)],
                        out.at[cid].at[pl.ds(r0, rows_per)])

    scratch = [
        pltpu.VMEM_SHARED((ACC, 128), jnp.float32),
        pltpu.VMEM((8, 2, CH), jnp.int32),
        pltpu.VMEM((CH, 128), jnp.float32),
        pltpu.VMEM((CH, 128), jnp.float32),
        pltpu.SemaphoreType.DMA,
        pltpu.SemaphoreType.DMA,
    ]
    return pl.kernel(body, out_type=jax.ShapeDtypeStruct((NC, ACC, 128),
                                                         jnp.float32),
                     mesh=mesh, scratch_types=scratch)


def _make_sc_cnt(ACC, ETC):
    NCHC = ETC // CH
    rows_per = ACC // NS
    mesh = plsc.VectorSubcoreMesh(core_axis_name="c", subcore_axis_name="s")

    def body(didx, zcnt, ones, cnt_out, cnt_sh, ibuf, ones_v):
        cid = lax.axis_index("c")
        sid = lax.axis_index("s")
        wid = cid * NS + sid
        r0 = sid * rows_per
        pltpu.sync_copy(zcnt.at[pl.ds(r0, rows_per)],
                        cnt_sh.at[pl.ds(r0, rows_per)])
        pltpu.sync_copy(ones, ones_v)
        plsc.subcore_barrier()

        def step(t, carry):
            pltpu.sync_copy(didx.at[wid].at[pl.ds(8 * t, 8)], ibuf)
            for q in range(8):
                pltpu.sync_copy(ones_v, cnt_sh.at[ibuf.at[q].at[0]],
                                add=True)
            return carry

        lax.fori_loop(0, NCHC // 8, step, 0)
        plsc.subcore_barrier()
        pltpu.sync_copy(cnt_sh.at[pl.ds(r0, rows_per)],
                        cnt_out.at[cid].at[pl.ds(r0, rows_per)])

    scratch = [
        pltpu.VMEM_SHARED((ACC, CH), jnp.float32),
        pltpu.VMEM((8, 1, CH), jnp.int32),
        pltpu.VMEM((CH, CH), jnp.float32),
    ]
    return pl.kernel(body, out_type=jax.ShapeDtypeStruct((NC, ACC, CH),
                                                         jnp.float32),
                     mesh=mesh, scratch_types=scratch)


def _stage_a_body(N, a0_ref, a1_ref, cnt_ref, root_ref, b_ref,
                  hpre_ref, stats_ref):
    i = pl.program_id(0)
    cnt = jnp.maximum(jnp.sum(cnt_ref[...], axis=(0, 2)) * (1.0 / 128.0), 1.0)
    h = (jnp.concatenate([a0_ref[0], a1_ref[0]], axis=1) / cnt[:, None]
         + root_ref[...] + b_ref[...])
    hpre_ref[...] = h.astype(jnp.bfloat16)
    s = jnp.concatenate([jnp.sum(h, axis=0, keepdims=True),
                         jnp.sum(h * h, axis=0, keepdims=True)], axis=0)

    @pl.when(i == 0)
    def _():
        stats_ref[...] = s

    @pl.when(i > 0)
    def _():
        stats_ref[...] += s


def _mm1_body(N, RH, hpre_ref, stats_ref, g_ref, be_ref, w_ref,
              tbl_ref, root_ref):
    s = stats_ref[...]
    mean = s[0:1] * (1.0 / N)
    var = s[1:2] * (1.0 / N) - mean * mean
    inv = lax.rsqrt(var + EPS) * g_ref[...]
    y = (hpre_ref[...].astype(jnp.float32) - mean) * inv + be_ref[...]
    xb = _elu(y).astype(jnp.bfloat16)
    acc = jnp.dot(xb, w_ref[...], preferred_element_type=jnp.float32)
    tbl_ref[0] = acc[:, :RH]
    tbl_ref[1] = acc[:, RH:2 * RH]
    root_ref[...] = acc[:, 2 * RH:2 * RH + 256]


def _mm1_call(hpre, stats, g0r, be0r, wcat, N, R):
    RH = R * 128
    KW = wcat.shape[1]
    return pl.pallas_call(
        functools.partial(_mm1_body, N, RH),
        grid=(N // MB,),
        in_specs=[
            pl.BlockSpec((MB, 256), lambda i: (i, 0)),
            pl.BlockSpec((2, 256), lambda i: (0, 0)),
            pl.BlockSpec((1, 256), lambda i: (0, 0)),
            pl.BlockSpec((1, 256), lambda i: (0, 0)),
            pl.BlockSpec((256, KW), lambda i: (0, 0)),
        ],
        out_specs=[
            pl.BlockSpec((NC, MB, RH), lambda i: (0, i, 0)),
            pl.BlockSpec((MB, 256), lambda i: (i, 0)),
        ],
        out_shape=[
            jax.ShapeDtypeStruct((NC, N, RH), jnp.float32),
            jax.ShapeDtypeStruct((N, 256), jnp.float32),
        ],
    )(hpre, stats, g0r, be0r, wcat)


def _stage_c_body(a0_ref, a1_ref, cnt_ref, root_ref, b_ref, skip_ref,
                  sb_ref, out_ref):
    cnt = jnp.maximum(jnp.sum(cnt_ref[...], axis=(0, 2)) * (1.0 / 128.0), 1.0)
    h = (jnp.concatenate([a0_ref[0], a1_ref[0]], axis=1) / cnt[:, None]
         + root_ref[...] + b_ref[...])
    h = _elu(h)
    h = h + skip_ref[...] + sb_ref[...]
    out_ref[...] = _elu(h)


def _half_spec(c):
    return pl.BlockSpec((1, MB, 128), lambda i, c=c: (c, i, 0))


def kernel(x, edge_index, edge_types, w0, root0, b0, w1, root1, b1,
           gamma0, beta0, skip_w, skip_b):
    N, F = x.shape
    R = w0.shape[0]
    E = edge_index.shape[1]

    NW = NC * NS
    ETP = -(-E // NS // (2 * IB * CH)) * (2 * IB * CH)
    EP = NS * ETP
    ACC = -(-(N + 1) // (NS * 8)) * (NS * 8)
    NCH = ETP // CH

    src = edge_index[0].astype(jnp.int32)
    dst = edge_index[1].astype(jnp.int32)
    ety = edge_types.astype(jnp.int32)
    pad = EP - E
    src_p = jnp.concatenate([src, jnp.zeros((pad,), jnp.int32)])
    ety_p = jnp.concatenate([ety, jnp.zeros((pad,), jnp.int32)])
    dst_p = jnp.concatenate([dst, jnp.full((pad,), N, jnp.int32)])
    gidx = (src_p * R + ety_p).reshape(NS, NCH, CH)
    dst2 = dst_p.reshape(NS, NCH, CH)
    idx_all = jnp.stack([gidx, dst2], axis=2)
    ETC = EP // NW
    didx = dst_p.reshape(NW, ETC // CH, 1, CH)
    zacc = jnp.zeros((ACC, 128), jnp.float32)
    zcnt = jnp.zeros((ACC, CH), jnp.float32)
    ones = jnp.ones((CH, CH), jnp.float32)

    def wcat_of(w, extra):
        h0 = w[:, :, :128].transpose(1, 0, 2).reshape(F, R * 128)
        h1 = w[:, :, 128:].transpose(1, 0, 2).reshape(F, R * 128)
        return jnp.concatenate([h0, h1] + extra, axis=1).astype(jnp.bfloat16)

    wcat0 = wcat_of(w0, [root0, skip_w])
    wcat1 = wcat_of(w1, [root1])
    xb = x.astype(jnp.bfloat16)
    b0r = b0.reshape(1, 256)
    b1r = b1.reshape(1, 256)
    g0r = gamma0.reshape(1, 256)
    be0r = beta0.reshape(1, 256)
    sbr = skip_b.reshape(1, 256)

    sc_agg = _make_sc_agg(N, R, ACC, ETP)
    sc_cnt = _make_sc_cnt(ACC, ETC)

    cnt_p = sc_cnt(didx, zcnt, ones)
    tbl0, xroot0, xskip = _mm_call(xb, wcat0, N, R, True)
    agg0 = sc_agg(tbl0.reshape(NC, N * R, 128), idx_all, zacc)

    grid = N // MB
    hpre, stats = pl.pallas_call(
        functools.partial(_stage_a_body, N),
        grid=(grid,),
        in_specs=[
            _half_spec(0),
            _half_spec(1),
            pl.BlockSpec((NC, MB, 128), lambda i: (0, i, 0)),
            pl.BlockSpec((MB, 256), lambda i: (i, 0)),
            pl.BlockSpec((1, 256), lambda i: (0, 0)),
        ],
        out_specs=[
            pl.BlockSpec((MB, 256), lambda i: (i, 0)),
            pl.BlockSpec((2, 256), lambda i: (0, 0)),
        ],
        out_shape=[
            jax.ShapeDtypeStruct((N, 256), jnp.bfloat16),
            jax.ShapeDtypeStruct((2, 256), jnp.float32),
        ],
    )(agg0, agg0, cnt_p, xroot0, b0r)

    tbl1, hroot1 = _mm1_call(hpre, stats, g0r, be0r, wcat1, N, R)
    agg1 = sc_agg(tbl1.reshape(NC, N * R, 128), idx_all, zacc)

    out = pl.pallas_call(
        _stage_c_body,
        grid=(grid,),
        in_specs=[
            _half_spec(0),
            _half_spec(1),
            pl.BlockSpec((NC, MB, 128), lambda i: (0, i, 0)),
            pl.BlockSpec((MB, 256), lambda i: (i, 0)),
            pl.BlockSpec((1, 256), lambda i: (0, 0)),
            pl.BlockSpec((MB, 256), lambda i: (i, 0)),
            pl.BlockSpec((1, 256), lambda i: (0, 0)),
        ],
        out_specs=pl.BlockSpec((MB, 256), lambda i: (i, 0)),
        out_shape=jax.ShapeDtypeStruct((N, 256), jnp.float32),
    )(agg1, agg1, cnt_p, hroot1, b1r, xskip, sbr)
    return out
